# baseline (speedup 1.0000x reference)
import jax
import jax.numpy as jnp
from jax.experimental import pallas as pl
from jax.experimental.pallas import tpu as pltpu

_BN = 2000
_NBUF = 3


def _cc_kernel(t_hbm, f_hbm, o_ref, cnt_ref):
    o_ref[...] = jnp.zeros_like(o_ref)
    cnt_ref[...] = jnp.zeros_like(cnt_ref)

    def _step(t_ref, f_ref):
        t = t_ref[...]
        f = f_ref[...]
        o_ref[...] += jax.lax.dot_general(
            t, f, (((0,), (0,)), ((), ())), preferred_element_type=jnp.float32
        )
        cnt_ref[...] += jnp.sum(t, axis=0, keepdims=True)

    n, c = t_hbm.shape
    d = f_hbm.shape[1]
    pltpu.emit_pipeline(
        _step,
        grid=(n // _BN,),
        in_specs=[
            pl.BlockSpec((_BN, c), lambda i: (i, 0),
                         pipeline_mode=pl.Buffered(buffer_count=_NBUF)),
            pl.BlockSpec((_BN, d), lambda i: (i, 0),
                         pipeline_mode=pl.Buffered(buffer_count=_NBUF)),
        ],
    )(t_hbm, f_hbm)

    eye = (
        jax.lax.broadcasted_iota(jnp.int32, (c, c), 0)
        == jax.lax.broadcasted_iota(jnp.int32, (c, c), 1)
    ).astype(jnp.float32)
    cnt_col = jax.lax.dot_general(
        eye, cnt_ref[...], (((1,), (1,)), ((), ())),
        preferred_element_type=jnp.float32,
    )
    o_ref[...] = o_ref[...] / cnt_col


def kernel(features, targets):
    n, d = features.shape
    _, c = targets.shape
    return pl.pallas_call(
        _cc_kernel,
        in_specs=[
            pl.BlockSpec(memory_space=pl.ANY),
            pl.BlockSpec(memory_space=pl.ANY),
        ],
        out_specs=pl.BlockSpec(memory_space=pltpu.MemorySpace.VMEM),
        out_shape=jax.ShapeDtypeStruct((c, d), jnp.float32),
        scratch_shapes=[pltpu.VMEM((1, c), jnp.float32)],
    )(targets, features)
